# baseline (speedup 1.0000x reference)
import functools

import jax
import jax.numpy as jnp
import numpy as np
from jax import lax
from jax.experimental import pallas as pl
from jax.experimental.pallas import tpu as pltpu
from jax.experimental.pallas import tpu_sc as plsc

B = 16384
D = 128
NC = 2
NS = 16
NW = NC * NS
BPW = B // NW

CH = 64
NCHT = BPW // CH
NCHK = 2 * NCHT
NBUF = 6
AHEAD = 3

_PERM = np.empty((D,), np.int64)
for _g in range(D // 32):
    for _t in range(16):
        _PERM[32 * _g + 2 * _t] = 32 * _g + _t
        _PERM[32 * _g + 2 * _t + 1] = 32 * _g + 16 + _t


def _pack_rows(fb, bb):
    def row(r, carry):
        for g in range(D // 32):
            lo = lax.bitcast_convert_type(fb[r, pl.ds(32 * g, 16)], jnp.int32)
            hi = lax.bitcast_convert_type(fb[r, pl.ds(32 * g + 16, 16)],
                                          jnp.int32)
            lw = ((lo + 0x7FFF + ((lo >> 16) & 1)) >> 16) & 0xFFFF
            hw = (hi + 0x7FFF + ((hi >> 16) & 1)) & ~0xFFFF
            bb[r, pl.ds(16 * g, 16)] = lw | hw
        return carry
    lax.fori_loop(0, CH, row, 0)


def _gather_body(xt_hbm, w_hbm, h_hbm, z_out,
                 idxu_v, idxv_v, *bufs_and_sems):
    fbufs = bufs_and_sems[:NBUF]
    bbufs = bufs_and_sems[NBUF:2 * NBUF]
    gsems = bufs_and_sems[2 * NBUF:3 * NBUF]
    wsems = bufs_and_sems[3 * NBUF:4 * NBUF]
    wid = lax.axis_index("s") * NC + lax.axis_index("c")
    base = wid * BPW
    pltpu.sync_copy(xt_hbm.at[0, pl.ds(base, BPW)], idxu_v)
    pltpu.sync_copy(xt_hbm.at[1, pl.ds(base, BPW)], idxv_v)

    def chunk(j):
        t, c = divmod(j, NCHT)
        idx = (idxu_v, idxv_v)[t]
        tab = (w_hbm, h_hbm)[t]
        return tab, idx.at[pl.ds(c * CH, CH)], t, base + c * CH

    def fire_gather(j):
        tab, idxsl, _, _ = chunk(j)
        return pltpu.async_copy(tab.at[idxsl], fbufs[j % NBUF],
                                gsems[j % NBUF])

    gd = [None] * NCHK
    wd = [None] * NCHK
    for j in range(min(AHEAD, NCHK)):
        gd[j] = fire_gather(j)
    for k in range(NCHK):
        j = k + AHEAD
        if j < NCHK:
            if j >= NBUF:
                wd[j - NBUF].wait()
            gd[j] = fire_gather(j)
        gd[k].wait()
        _pack_rows(fbufs[k % NBUF], bbufs[k % NBUF])
        _, _, t, off = chunk(k)
        wd[k] = pltpu.async_copy(bbufs[k % NBUF],
                                 z_out.at[t, pl.ds(off, CH)],
                                 wsems[k % NBUF])
    for k in range(max(0, NCHK - NBUF), NCHK):
        wd[k].wait()


@functools.cache
def _gather():
    return pl.kernel(
        _gather_body,
        mesh=plsc.VectorSubcoreMesh(core_axis_name="c", subcore_axis_name="s"),
        out_type=[
            jax.ShapeDtypeStruct((2, B, D // 2), jnp.int32),
        ],
        scratch_types=(
            [pltpu.VMEM((BPW,), jnp.int32),
             pltpu.VMEM((BPW,), jnp.int32)]
            + [pltpu.VMEM((CH, D), jnp.float32) for _ in range(NBUF)]
            + [pltpu.VMEM((CH, D // 2), jnp.int32) for _ in range(NBUF)]
            + [pltpu.SemaphoreType.DMA for _ in range(2 * NBUF)]
        ),
    )


BLK = 2048


def _mlp_body(u_ref, v_ref, a_ref, bm_ref, b1_ref, w2_ref, o_ref):
    u = u_ref[0].astype(jnp.float32)
    v = v_ref[0].astype(jnp.float32)
    h = jnp.dot(u, a_ref[:], preferred_element_type=jnp.float32)
    h = h + jnp.dot(v, bm_ref[:], preferred_element_type=jnp.float32)
    h = jnp.maximum(h + b1_ref[:][None, :], 0.0)
    o_ref[:] = jnp.dot(h, w2_ref[:], preferred_element_type=jnp.float32)


def _mlp(z, a, bm, b1, w2pad):
    return pl.pallas_call(
        _mlp_body,
        grid=(B // BLK,),
        in_specs=[
            pl.BlockSpec((1, BLK, D), lambda i: (0, i, 0)),
            pl.BlockSpec((1, BLK, D), lambda i: (1, i, 0)),
            pl.BlockSpec((D, D), lambda i: (0, 0)),
            pl.BlockSpec((D, D), lambda i: (0, 0)),
            pl.BlockSpec((D,), lambda i: (0,)),
            pl.BlockSpec((D, D), lambda i: (0, 0)),
        ],
        out_specs=pl.BlockSpec((BLK, D), lambda i: (i, 0)),
        out_shape=jax.ShapeDtypeStruct((B, D), jnp.float32),
        compiler_params=pltpu.CompilerParams(
            dimension_semantics=("arbitrary",),
        ),
    )(z, z, a, bm, b1, w2pad)


def kernel(x, W, H, W1, b1, W2):
    xt = x.T
    (z32,) = _gather()(xt, W, H)
    z = lax.bitcast_convert_type(z32, jnp.bfloat16).reshape(2, B, D)
    perm = jnp.asarray(_PERM)
    a = W1[:, :D].T[perm]
    bm = W1[:, D:].T[perm]
    w2pad = jnp.zeros((D, D), jnp.float32).at[:, 0].set(W2[0])
    out = _mlp(z, a, bm, b1, w2pad)
    return out[:, :1]


# speedup vs baseline: 2.0923x; 2.0923x over previous
import functools

import jax
import jax.numpy as jnp
from jax import lax
from jax.experimental import pallas as pl
from jax.experimental.pallas import tpu as pltpu
from jax.experimental.pallas import tpu_sc as plsc

B = 16384
D = 128
NC = 2
NS = 16
NW = NC * NS
BPW = B // NW

CH = 64
NCHT = BPW // CH
NCHK = 2 * NCHT
NBUF = 12
AHEAD = 6


def _gather_body(xt_hbm, w_hbm, h_hbm, z_out,
                 idxu_v, idxv_v, *bufs_and_sems):
    fbufs = bufs_and_sems[:NBUF]
    gsems = bufs_and_sems[NBUF:2 * NBUF]
    wsems = bufs_and_sems[2 * NBUF:3 * NBUF]
    wid = lax.axis_index("s") * NC + lax.axis_index("c")
    base = wid * BPW
    pltpu.sync_copy(xt_hbm.at[0, pl.ds(base, BPW)], idxu_v)
    pltpu.sync_copy(xt_hbm.at[1, pl.ds(base, BPW)], idxv_v)

    def chunk(j):
        t, c = divmod(j, NCHT)
        idx = (idxu_v, idxv_v)[t]
        tab = (w_hbm, h_hbm)[t]
        return tab, idx.at[pl.ds(c * CH, CH)], t, base + c * CH

    def fire_gather(j):
        tab, idxsl, _, _ = chunk(j)
        return pltpu.async_copy(tab.at[idxsl], fbufs[j % NBUF],
                                gsems[j % NBUF])

    gd = [None] * NCHK
    wd = [None] * NCHK
    for j in range(min(AHEAD, NCHK)):
        gd[j] = fire_gather(j)
    for k in range(NCHK):
        j = k + AHEAD
        if j < NCHK:
            if j >= NBUF:
                wd[j - NBUF].wait()
            gd[j] = fire_gather(j)
        gd[k].wait()
        _, _, t, off = chunk(k)
        wd[k] = pltpu.async_copy(fbufs[k % NBUF],
                                 z_out.at[t, pl.ds(off, CH)],
                                 wsems[k % NBUF])
    for k in range(max(0, NCHK - NBUF), NCHK):
        wd[k].wait()


@functools.cache
def _gather():
    return pl.kernel(
        _gather_body,
        mesh=plsc.VectorSubcoreMesh(core_axis_name="c", subcore_axis_name="s"),
        out_type=[
            jax.ShapeDtypeStruct((2, B, D), jnp.float32),
        ],
        scratch_types=(
            [pltpu.VMEM((BPW,), jnp.int32),
             pltpu.VMEM((BPW,), jnp.int32)]
            + [pltpu.VMEM((CH, D), jnp.float32) for _ in range(NBUF)]
            + [pltpu.SemaphoreType.DMA for _ in range(2 * NBUF)]
        ),
    )


BLK = 2048


def _mlp_body(u_ref, v_ref, a_ref, bm_ref, b1_ref, w2_ref, o_ref):
    h = jnp.dot(u_ref[0], a_ref[:], preferred_element_type=jnp.float32)
    h = h + jnp.dot(v_ref[0], bm_ref[:], preferred_element_type=jnp.float32)
    h = jnp.maximum(h + b1_ref[:][None, :], 0.0)
    o_ref[:] = jnp.dot(h, w2_ref[:], preferred_element_type=jnp.float32)


def _mlp(z, a, bm, b1, w2pad):
    return pl.pallas_call(
        _mlp_body,
        grid=(B // BLK,),
        in_specs=[
            pl.BlockSpec((1, BLK, D), lambda i: (0, i, 0)),
            pl.BlockSpec((1, BLK, D), lambda i: (1, i, 0)),
            pl.BlockSpec((D, D), lambda i: (0, 0)),
            pl.BlockSpec((D, D), lambda i: (0, 0)),
            pl.BlockSpec((D,), lambda i: (0,)),
            pl.BlockSpec((D, D), lambda i: (0, 0)),
        ],
        out_specs=pl.BlockSpec((BLK, D), lambda i: (i, 0)),
        out_shape=jax.ShapeDtypeStruct((B, D), jnp.float32),
        compiler_params=pltpu.CompilerParams(
            dimension_semantics=("arbitrary",),
        ),
    )(z, z, a, bm, b1, w2pad)


def kernel(x, W, H, W1, b1, W2):
    xt = x.T
    (z,) = _gather()(xt, W, H)
    a = W1[:, :D].T
    bm = W1[:, D:].T
    w2pad = jnp.zeros((D, D), jnp.float32).at[:, 0].set(W2[0])
    out = _mlp(z, a, bm, b1, w2pad)
    return out[:, :1]
